# baseline (speedup 1.0000x reference)
import functools

import jax
import jax.numpy as jnp
from jax.experimental import pallas as pl
from jax.experimental.pallas import tpu as pltpu

_EPS = 1e-5


def _pass1_kernel(x_ref, w1_ref, adj_ref, s1_ref, sh1_ref, w2_ref,
                  t_ref, support_ref):
    @pl.when(pl.program_id(0) == 0)
    def _():
        support_ref[...] = jnp.dot(x_ref[...], w1_ref[...],
                                   preferred_element_type=jnp.float32)

    acc = jnp.dot(adj_ref[...], support_ref[...],
                  preferred_element_type=jnp.float32)
    h1 = jnp.maximum(acc * s1_ref[...] + sh1_ref[...], 0.0)
    t_ref[...] = jnp.dot(h1, w2_ref[...], preferred_element_type=jnp.float32)


def _pass2_kernel(adj_ref, t_ref, s2_ref, sh2_ref, dw_ref, db_ref,
                  mu_ref, out_ref):
    acc = jnp.dot(adj_ref[...], t_ref[...],
                  preferred_element_type=jnp.float32)
    mu = acc * s2_ref[...] + sh2_ref[...]
    mu_ref[...] = mu
    out_ref[...] = jnp.dot(mu, dw_ref[...],
                           preferred_element_type=jnp.float32) + db_ref[...]


def kernel(x, adj, W1, W2, g1, b1, m1, v1, g2, b2, m2, v2, dec_W, dec_b):
    N, F = x.shape
    H1 = W1.shape[1]
    H2 = W2.shape[1]
    C = dec_W.shape[0]

    s1 = (g1 / jnp.sqrt(v1 + _EPS)).reshape(1, H1)
    sh1 = (b1 - m1 * g1 / jnp.sqrt(v1 + _EPS)).reshape(1, H1)
    s2 = (g2 / jnp.sqrt(v2 + _EPS)).reshape(1, H2)
    sh2 = (b2 - m2 * g2 / jnp.sqrt(v2 + _EPS)).reshape(1, H2)
    dwT = dec_W.T
    db = dec_b.reshape(1, C)

    BM = 400
    grid = (N // BM,)

    t = pl.pallas_call(
        _pass1_kernel,
        grid=grid,
        in_specs=[
            pl.BlockSpec((N, F), lambda i: (0, 0)),
            pl.BlockSpec((F, H1), lambda i: (0, 0)),
            pl.BlockSpec((BM, N), lambda i: (i, 0)),
            pl.BlockSpec((1, H1), lambda i: (0, 0)),
            pl.BlockSpec((1, H1), lambda i: (0, 0)),
            pl.BlockSpec((H1, H2), lambda i: (0, 0)),
        ],
        out_specs=pl.BlockSpec((BM, H2), lambda i: (i, 0)),
        out_shape=jax.ShapeDtypeStruct((N, H2), jnp.float32),
        scratch_shapes=[pltpu.VMEM((N, H1), jnp.float32)],
    )(x, W1, adj, s1, sh1, W2)

    mu, out = pl.pallas_call(
        _pass2_kernel,
        grid=grid,
        in_specs=[
            pl.BlockSpec((BM, N), lambda i: (i, 0)),
            pl.BlockSpec((N, H2), lambda i: (0, 0)),
            pl.BlockSpec((1, H2), lambda i: (0, 0)),
            pl.BlockSpec((1, H2), lambda i: (0, 0)),
            pl.BlockSpec((H2, C), lambda i: (0, 0)),
            pl.BlockSpec((1, C), lambda i: (0, 0)),
        ],
        out_specs=[
            pl.BlockSpec((BM, H2), lambda i: (i, 0)),
            pl.BlockSpec((BM, C), lambda i: (i, 0)),
        ],
        out_shape=[
            jax.ShapeDtypeStruct((N, H2), jnp.float32),
            jax.ShapeDtypeStruct((N, C), jnp.float32),
        ],
    )(adj, t, s2, sh2, dwT, db)

    return (out, mu, mu)
